# baseline (speedup 1.0000x reference)
import functools

import jax
import jax.numpy as jnp
from jax import lax
from jax.experimental import pallas as pl
from jax.experimental.pallas import tpu as pltpu
from jax.experimental.pallas import tpu_sc as plsc

_V = 1000000
_D = 64
_B = 16384

_info = plsc.get_sparse_core_info()
_NC = _info.num_cores
_NS = _info.num_subcores
_NW = _NC * _NS
_BPW = _B // _NW
_CHUNK = 128
_NCHUNK = _BPW // _CHUNK


def _glove_kernel(left_hbm, right_hbm, l_emb_hbm, l_bias_hbm, r_emb_hbm,
                  r_bias_hbm, out_hbm,
                  lidx_v, ridx_v, lrows_v, rrows_v, lb_v, rb_v, out_v, sem):
    wid = lax.axis_index("s") * _NC + lax.axis_index("c")
    base = wid * _BPW

    pltpu.sync_copy(left_hbm.at[wid], lidx_v)
    pltpu.sync_copy(right_hbm.at[wid], ridx_v)

    copies = []
    for j in range(_NCHUNK):
        dst = pl.ds(j * _CHUNK, _CHUNK)
        copies.append(pltpu.async_copy(
            l_emb_hbm.at[lidx_v.at[j]], lrows_v.at[dst], sem))
        copies.append(pltpu.async_copy(
            r_emb_hbm.at[ridx_v.at[j]], rrows_v.at[dst], sem))
        copies.append(pltpu.async_copy(
            l_bias_hbm.at[lidx_v.at[j]], lb_v.at[dst], sem))
        copies.append(pltpu.async_copy(
            r_bias_hbm.at[ridx_v.at[j]], rb_v.at[dst], sem))
    for c in copies:
        c.wait()

    iota16 = lax.iota(jnp.int32, 16)

    def group_body(g, _):
        i0 = g * 16
        dots = jnp.zeros((16,), jnp.float32)
        for k in range(16):
            i = i0 + k
            acc = lrows_v[i, pl.ds(0, 16)] * rrows_v[i, pl.ds(0, 16)]
            for c in range(1, _D // 16):
                acc = acc + (lrows_v[i, pl.ds(16 * c, 16)]
                             * rrows_v[i, pl.ds(16 * c, 16)])
            dots = jnp.where(iota16 == k, jnp.sum(acc), dots)
        out_v[pl.ds(i0, 16)] = dots + lb_v[pl.ds(i0, 16)] + rb_v[pl.ds(i0, 16)]
        return 0

    lax.fori_loop(0, _BPW // 16, group_body, 0)

    pltpu.sync_copy(out_v, out_hbm.at[pl.ds(base, _BPW)])


@functools.partial(jax.jit, donate_argnums=())
def kernel(left, right, l_emb, l_bias, r_emb, r_bias):
    mesh = plsc.VectorSubcoreMesh(core_axis_name="c", subcore_axis_name="s")
    left_r = left.reshape(_NW, _NCHUNK, _CHUNK)
    right_r = right.reshape(_NW, _NCHUNK, _CHUNK)
    l_bias_f = l_bias.reshape(_V)
    r_bias_f = r_bias.reshape(_V)
    run = pl.kernel(
        _glove_kernel,
        mesh=mesh,
        out_type=jax.ShapeDtypeStruct((_B,), jnp.float32),
        compiler_params=pltpu.CompilerParams(
            needs_layout_passes=False, use_tc_tiling_on_sc=False),
        scratch_types=[
            pltpu.VMEM((_NCHUNK, _CHUNK), jnp.int32),
            pltpu.VMEM((_NCHUNK, _CHUNK), jnp.int32),
            pltpu.VMEM((_BPW, _D), jnp.float32),
            pltpu.VMEM((_BPW, _D), jnp.float32),
            pltpu.VMEM((_BPW,), jnp.float32),
            pltpu.VMEM((_BPW,), jnp.float32),
            pltpu.VMEM((_BPW,), jnp.float32),
            pltpu.SemaphoreType.DMA,
        ],
    )
    return run(left_r, right_r, l_emb, l_bias_f, r_emb, r_bias_f)
